# baseline (speedup 1.0000x reference)
import functools

import jax
import jax.numpy as jnp
from jax import lax
from jax.experimental import pallas as pl
from jax.experimental.pallas import tpu as pltpu
from jax.experimental.pallas import tpu_sc as plsc

N_SC = 2
N_TILE = 16
NW = N_SC * N_TILE

CH = 80


def _sc_segment_sums(node_features, senders_r, receivers_r, edges_r,
                     zA, zE, zD, ones, n_pad, nch):
    n_nodes, d_feat = node_features.shape
    d_edge = edges_r.shape[-1]
    rows_per_tile = n_pad // N_TILE
    nz = rows_per_tile // CH

    mesh = plsc.VectorSubcoreMesh(core_axis_name="c", subcore_axis_name="s")

    @functools.partial(
        pl.kernel,
        out_type=[
            jax.ShapeDtypeStruct((N_SC, n_pad, d_feat), jnp.float32),
            jax.ShapeDtypeStruct((N_SC, n_pad, d_edge), jnp.float32),
            jax.ShapeDtypeStruct((N_SC, n_pad), jnp.float32),
        ],
        mesh=mesh,
        scratch_types=[
            pltpu.VMEM((CH,), jnp.int32),
            pltpu.VMEM((CH,), jnp.int32),
            pltpu.VMEM((CH, d_feat), jnp.float32),
            pltpu.VMEM((CH, d_edge), jnp.float32),
            pltpu.VMEM((CH,), jnp.float32),
            pltpu.VMEM_SHARED((n_pad, d_feat), jnp.float32),
            pltpu.VMEM_SHARED((n_pad, d_edge), jnp.float32),
            pltpu.VMEM_SHARED((n_pad,), jnp.float32),
            pltpu.SemaphoreType.DMA,
        ],
        compiler_params=pltpu.CompilerParams(use_tc_tiling_on_sc=False),
    )
    def sc_kernel(node_hbm, s_hbm, r_hbm, e_hbm, zA_hbm, zE_hbm, zD_hbm,
                  ones_hbm, out_raw, out_e, out_deg,
                  sidx, ridx, rows, erows, ones_v,
                  acc_raw, acc_e, acc_deg, sem):
        c = lax.axis_index("c")
        s = lax.axis_index("s")
        wid = c * N_TILE + s
        base = s * rows_per_tile

        pltpu.sync_copy(zA_hbm, rows)
        pltpu.sync_copy(zE_hbm, erows)
        pltpu.sync_copy(zD_hbm, ones_v)
        for k in range(nz):
            pltpu.sync_copy(rows, acc_raw.at[pl.ds(base + k * CH, CH)])
            pltpu.sync_copy(erows, acc_e.at[pl.ds(base + k * CH, CH)])
            pltpu.sync_copy(ones_v, acc_deg.at[pl.ds(base + k * CH, CH)])

        pltpu.sync_copy(ones_hbm, ones_v)
        ebase = wid * (nch * CH)

        plsc.subcore_barrier()

        def body(j, carry):
            off = ebase + j * CH
            pltpu.sync_copy(s_hbm.at[pl.ds(off, CH)], sidx)
            pltpu.sync_copy(r_hbm.at[pl.ds(off, CH)], ridx)
            pltpu.async_copy(node_hbm.at[sidx], rows, sem).wait()
            pltpu.sync_copy(rows, acc_raw.at[ridx], add=True)
            pltpu.sync_copy(e_hbm.at[pl.ds(off, CH)], erows)
            pltpu.sync_copy(erows, acc_e.at[ridx], add=True)
            pltpu.sync_copy(ones_v, acc_deg.at[ridx], add=True)
            return carry

        lax.fori_loop(0, nch, body, jnp.int32(0))

        plsc.subcore_barrier()

        pltpu.sync_copy(acc_raw.at[pl.ds(base, rows_per_tile)],
                        out_raw.at[c].at[pl.ds(base, rows_per_tile)])
        pltpu.sync_copy(acc_e.at[pl.ds(base, rows_per_tile)],
                        out_e.at[c].at[pl.ds(base, rows_per_tile)])
        pltpu.sync_copy(acc_deg.at[pl.ds(base, rows_per_tile)],
                        out_deg.at[c].at[pl.ds(base, rows_per_tile)])

    return sc_kernel(node_features, senders_r, receivers_r, edges_r,
                     zA, zE, zD, ones)


def _tc_combine(node_features, W, b2, We, be2, raw_p, se_p, deg_p):
    n_nodes, d_feat = node_features.shape
    d_out = W.shape[1] // 2
    d_edge = We.shape[0]
    blk = 2048
    grid = ((n_nodes + blk - 1) // blk,)

    def tc_kernel(x_ref, w_ref, b_ref, we_ref, be_ref, raw_ref, se_ref,
                  deg_ref, out_ref):
        w = w_ref[...]
        xi = jnp.dot(x_ref[...], w[:, :d_out],
                     preferred_element_type=jnp.float32)
        raw = raw_ref[0] + raw_ref[1]
        se = se_ref[0] + se_ref[1]
        deg = (deg_ref[0] + deg_ref[1])[:, None]
        num = jnp.dot(raw, w[:, d_out:], preferred_element_type=jnp.float32)
        num = num + jnp.dot(se, we_ref[...],
                            preferred_element_type=jnp.float32)
        num = num + deg * (b_ref[:, d_out:] + be_ref[...])
        out_ref[...] = xi + b_ref[:, :d_out] + num / (deg + 1e-6)

    return pl.pallas_call(
        tc_kernel,
        grid=grid,
        in_specs=[
            pl.BlockSpec((blk, d_feat), lambda i: (i, 0)),
            pl.BlockSpec((d_feat, 2 * d_out), lambda i: (0, 0)),
            pl.BlockSpec((1, 2 * d_out), lambda i: (0, 0)),
            pl.BlockSpec((d_edge, d_out), lambda i: (0, 0)),
            pl.BlockSpec((1, d_out), lambda i: (0, 0)),
            pl.BlockSpec((N_SC, blk, d_out), lambda i: (0, i, 0)),
            pl.BlockSpec((N_SC, blk, d_edge), lambda i: (0, i, 0)),
            pl.BlockSpec((N_SC, blk), lambda i: (0, i)),
        ],
        out_specs=pl.BlockSpec((blk, d_out), lambda i: (i, 0)),
        out_shape=jax.ShapeDtypeStruct((n_nodes, d_out), jnp.float32),
    )(node_features, W, b2, We, be2, raw_p, se_p, deg_p)


def kernel(node_features, senders, receivers, edge_features, W, b, We, be):
    n_nodes, d_feat = node_features.shape
    e = senders.shape[0]
    d_edge = edge_features.shape[1]

    e_per_w = e // NW
    nch = e_per_w // CH
    rpt = (n_nodes + N_TILE - 1) // N_TILE
    rows_per_tile = ((rpt + CH - 1) // CH) * CH
    n_pad = rows_per_tile * N_TILE

    senders_r = senders
    receivers_r = receivers
    edges_r = edge_features

    zA = jnp.zeros((CH, d_feat), jnp.float32)
    zE = jnp.zeros((CH, d_edge), jnp.float32)
    zD = jnp.zeros((CH,), jnp.float32)
    ones = jnp.ones((CH,), jnp.float32)

    raw_p, se_p, deg_p = _sc_segment_sums(
        node_features, senders_r, receivers_r, edges_r, zA, zE, zD, ones,
        n_pad, nch)

    return _tc_combine(node_features, W, b.reshape(1, -1), We,
                       be.reshape(1, -1), raw_p, se_p, deg_p)


# speedup vs baseline: 1.6630x; 1.6630x over previous
import functools

import jax
import jax.numpy as jnp
from jax import lax
from jax.experimental import pallas as pl
from jax.experimental.pallas import tpu as pltpu
from jax.experimental.pallas import tpu_sc as plsc

N_SC = 2
N_TILE = 16
NW = N_SC * N_TILE

CH = 80


def _sc_segment_sums(node_features, senders_r, receivers_r, edges_r,
                     zA, zE, zD, ones, n_pad, nch):
    n_nodes, d_feat = node_features.shape
    d_edge = edges_r.shape[-1]
    rows_per_tile = n_pad // N_TILE
    nz = rows_per_tile // CH

    mesh = plsc.VectorSubcoreMesh(core_axis_name="c", subcore_axis_name="s")

    @functools.partial(
        pl.kernel,
        out_type=[
            jax.ShapeDtypeStruct((N_SC, n_pad, d_feat), jnp.float32),
            jax.ShapeDtypeStruct((N_SC, n_pad, d_edge), jnp.float32),
            jax.ShapeDtypeStruct((N_SC, n_pad), jnp.float32),
        ],
        mesh=mesh,
        scratch_types=[
            pltpu.VMEM((nch * CH,), jnp.int32),
            [pltpu.VMEM((CH,), jnp.int32) for _ in range(2)],
            [pltpu.VMEM((CH, d_feat), jnp.float32) for _ in range(2)],
            [pltpu.VMEM((CH, d_edge), jnp.float32) for _ in range(2)],
            pltpu.VMEM((CH,), jnp.float32),
            pltpu.VMEM_SHARED((n_pad, d_feat), jnp.float32),
            pltpu.VMEM_SHARED((n_pad, d_edge), jnp.float32),
            pltpu.VMEM_SHARED((n_pad,), jnp.float32),
            [pltpu.SemaphoreType.DMA for _ in range(6)],
        ],
        compiler_params=pltpu.CompilerParams(use_tc_tiling_on_sc=False),
    )
    def sc_kernel(node_hbm, s_hbm, r_hbm, e_hbm, zA_hbm, zE_hbm, zD_hbm,
                  ones_hbm, out_raw, out_e, out_deg,
                  sidx_all, ridx, rows, erows, ones_v,
                  acc_raw, acc_e, acc_deg, sems):
        c = lax.axis_index("c")
        s = lax.axis_index("s")
        wid = c * N_TILE + s
        base = s * rows_per_tile
        gsem = sems[0:2]
        esem = sems[2:4]
        ssem = sems[4:6]

        pltpu.sync_copy(zA_hbm, rows[0])
        pltpu.sync_copy(zE_hbm, erows[0])
        pltpu.sync_copy(zD_hbm, ones_v)
        for k in range(nz):
            pltpu.sync_copy(rows[0], acc_raw.at[pl.ds(base + k * CH, CH)])
            pltpu.sync_copy(erows[0], acc_e.at[pl.ds(base + k * CH, CH)])
            pltpu.sync_copy(ones_v, acc_deg.at[pl.ds(base + k * CH, CH)])

        pltpu.sync_copy(ones_hbm, ones_v)
        ebase = wid * (nch * CH)
        pltpu.sync_copy(s_hbm.at[pl.ds(ebase, nch * CH)], sidx_all)

        plsc.subcore_barrier()

        def start(j, p):
            off = ebase + j * CH
            pltpu.sync_copy(r_hbm.at[pl.ds(off, CH)], ridx[p])
            g = pltpu.async_copy(node_hbm.at[sidx_all.at[pl.ds(j * CH, CH)]],
                                 rows[p], gsem[p])
            el = pltpu.async_copy(e_hbm.at[pl.ds(off, CH)], erows[p], esem[p])
            return g, el

        def scatter(p):
            s1 = pltpu.async_copy(rows[p], acc_raw.at[ridx[p]], ssem[p],
                                  add=True)
            s2 = pltpu.async_copy(erows[p], acc_e.at[ridx[p]], ssem[p],
                                  add=True)
            s3 = pltpu.async_copy(ones_v, acc_deg.at[ridx[p]], ssem[p],
                                  add=True)
            return s1, s2, s3

        def pair_body(jj, carry):
            g0, e0 = start(2 * jj, 0)
            g1, e1 = start(2 * jj + 1, 1)
            g0.wait()
            e0.wait()
            sc0 = scatter(0)
            g1.wait()
            e1.wait()
            sc1 = scatter(1)
            for d in sc0 + sc1:
                d.wait()
            return carry

        lax.fori_loop(0, nch // 2, pair_body, jnp.int32(0))

        for j in range(nch - (nch % 2), nch):
            g, el = start(j, 0)
            g.wait()
            el.wait()
            for d in scatter(0):
                d.wait()

        plsc.subcore_barrier()

        pltpu.sync_copy(acc_raw.at[pl.ds(base, rows_per_tile)],
                        out_raw.at[c].at[pl.ds(base, rows_per_tile)])
        pltpu.sync_copy(acc_e.at[pl.ds(base, rows_per_tile)],
                        out_e.at[c].at[pl.ds(base, rows_per_tile)])
        pltpu.sync_copy(acc_deg.at[pl.ds(base, rows_per_tile)],
                        out_deg.at[c].at[pl.ds(base, rows_per_tile)])

    return sc_kernel(node_features, senders_r, receivers_r, edges_r,
                     zA, zE, zD, ones)


def _tc_combine(node_features, W, b2, We, be2, raw_p, se_p, deg_p):
    n_nodes, d_feat = node_features.shape
    d_out = W.shape[1] // 2
    d_edge = We.shape[0]
    blk = 2048
    grid = ((n_nodes + blk - 1) // blk,)

    def tc_kernel(x_ref, w_ref, b_ref, we_ref, be_ref, raw_ref, se_ref,
                  deg_ref, out_ref):
        w = w_ref[...]
        xi = jnp.dot(x_ref[...], w[:, :d_out],
                     preferred_element_type=jnp.float32)
        raw = raw_ref[0] + raw_ref[1]
        se = se_ref[0] + se_ref[1]
        deg = (deg_ref[0] + deg_ref[1])[:, None]
        num = jnp.dot(raw, w[:, d_out:], preferred_element_type=jnp.float32)
        num = num + jnp.dot(se, we_ref[...],
                            preferred_element_type=jnp.float32)
        num = num + deg * (b_ref[:, d_out:] + be_ref[...])
        out_ref[...] = xi + b_ref[:, :d_out] + num / (deg + 1e-6)

    return pl.pallas_call(
        tc_kernel,
        grid=grid,
        in_specs=[
            pl.BlockSpec((blk, d_feat), lambda i: (i, 0)),
            pl.BlockSpec((d_feat, 2 * d_out), lambda i: (0, 0)),
            pl.BlockSpec((1, 2 * d_out), lambda i: (0, 0)),
            pl.BlockSpec((d_edge, d_out), lambda i: (0, 0)),
            pl.BlockSpec((1, d_out), lambda i: (0, 0)),
            pl.BlockSpec((N_SC, blk, d_out), lambda i: (0, i, 0)),
            pl.BlockSpec((N_SC, blk, d_edge), lambda i: (0, i, 0)),
            pl.BlockSpec((N_SC, blk), lambda i: (0, i)),
        ],
        out_specs=pl.BlockSpec((blk, d_out), lambda i: (i, 0)),
        out_shape=jax.ShapeDtypeStruct((n_nodes, d_out), jnp.float32),
    )(node_features, W, b2, We, be2, raw_p, se_p, deg_p)


def kernel(node_features, senders, receivers, edge_features, W, b, We, be):
    n_nodes, d_feat = node_features.shape
    e = senders.shape[0]
    d_edge = edge_features.shape[1]

    e_per_w = e // NW
    nch = e_per_w // CH
    rpt = (n_nodes + N_TILE - 1) // N_TILE
    rows_per_tile = ((rpt + CH - 1) // CH) * CH
    n_pad = rows_per_tile * N_TILE

    senders_r = senders
    receivers_r = receivers
    edges_r = edge_features

    zA = jnp.zeros((CH, d_feat), jnp.float32)
    zE = jnp.zeros((CH, d_edge), jnp.float32)
    zD = jnp.zeros((CH,), jnp.float32)
    ones = jnp.ones((CH,), jnp.float32)

    raw_p, se_p, deg_p = _sc_segment_sums(
        node_features, senders_r, receivers_r, edges_r, zA, zE, zD, ones,
        n_pad, nch)

    return _tc_combine(node_features, W, b.reshape(1, -1), We,
                       be.reshape(1, -1), raw_p, se_p, deg_p)
